# baseline (speedup 1.0000x reference)
import jax
import jax.numpy as jnp
from jax import lax
from jax.experimental import pallas as pl
from jax.experimental.pallas import tpu as pltpu
from jax.experimental.pallas import tpu_sc as plsc

D = 128
DH = D // 2
NSEG = 10000
NSEG_PAD = 10240
NC = 2
NS = 16
SB_ROWS = 512
NSB = 625
SLICE = NSEG_PAD // NS
CHUNK = 128
NCH = SB_ROWS // CHUNK
NBUF = 2


def _zero_fill(ref, nrows, ncols):
  z = jnp.zeros((16,), jnp.float32)
  def body(i, c):
    for j in range(ncols // 16):
      ref[i, pl.ds(j * 16, 16)] = z
    return c
  lax.fori_loop(0, nrows, body, 0)


def _sc_body(data_hbm, ids_hbm, out_hbm,
             buf0, buf1, buf2, idb0, idb1, idb2, ones_v, zcnt, accum, caccum,
             fsem0, fsem1, fsem2, ssem):
  cid = lax.axis_index("c")
  sid = lax.axis_index("s")
  bufs = (buf0, buf1, buf2)
  idbs = (idb0, idb1, idb2)
  fsems = (fsem0, fsem1, fsem2)

  s0 = (NSB * sid) // NS
  s_end = (NSB * (sid + 1)) // NS

  _zero_fill(buf0, SB_ROWS, DH)
  _zero_fill(zcnt, zcnt.shape[0], 16)
  one = jnp.ones((16,), jnp.float32)
  def ones_body(i, c):
    ones_v[i, :] = one
    return c
  lax.fori_loop(0, ones_v.shape[0], ones_body, 0)

  zbase = sid * SLICE
  pltpu.sync_copy(buf0, accum.at[pl.ds(zbase, SB_ROWS)])
  pltpu.sync_copy(buf0.at[pl.ds(0, SLICE - SB_ROWS)],
                  accum.at[pl.ds(zbase + SB_ROWS, SLICE - SB_ROWS)])
  pltpu.sync_copy(zcnt, caccum.at[pl.ds(zbase, SLICE)])
  plsc.subcore_barrier()

  col = cid * DH

  def fill(s, b):
    pltpu.async_copy(
        data_hbm.at[pl.ds(s * SB_ROWS, SB_ROWS), pl.ds(col, DH)],
        bufs[b], fsems[b])
    pltpu.async_copy(ids_hbm.at[pl.ds(s * NCH, NCH)], idbs[b], fsems[b])

  def fill_wait(s, b):
    pltpu.make_async_copy(
        data_hbm.at[pl.ds(s * SB_ROWS, SB_ROWS), pl.ds(col, DH)],
        bufs[b], fsems[b]).wait()
    pltpu.make_async_copy(ids_hbm.at[pl.ds(s * NCH, NCH)],
                          idbs[b], fsems[b]).wait()

  def scatter_start(b):
    descs = []
    for j in range(NCH):
      descs.append(pltpu.async_copy(
          bufs[b].at[pl.ds(j * CHUNK, CHUNK)], accum.at[idbs[b].at[j, 0]],
          ssem, add=True))
      descs.append(pltpu.async_copy(
          ones_v, caccum.at[idbs[b].at[j, 0]], ssem, add=True))
    return descs

  for b in range(NBUF - 1):
    @pl.when(s0 + b < s_end)
    def _(b=b):
      fill(s0 + b, b)

  def step(k, b):
    s = s0 + k
    @pl.when(s < s_end)
    def _():
      fill_wait(s, b)
      descs = scatter_start(b)
      @pl.when(s + NBUF - 1 < s_end)
      def _():
        fill(s + NBUF - 1, (b + NBUF - 1) % NBUF)
      for d in descs:
        d.wait()

  def pipe_body(i, c):
    for b in range(NBUF):
      step(i * NBUF + b, b)
    return c
  max_sb = (NSB + NS - 1) // NS
  lax.fori_loop(0, (max_sb + NBUF - 1) // NBUF, pipe_body, 0)
  plsc.subcore_barrier()

  pltpu.sync_copy(caccum.at[pl.ds(sid * SLICE, SLICE)], zcnt)
  for (off, sz) in ((0, SB_ROWS), (SB_ROWS, SLICE - SB_ROWS)):
    pltpu.sync_copy(accum.at[pl.ds(sid * SLICE + off, sz)],
                    buf0.at[pl.ds(0, sz)])
    def fin_body(r, c, off=off):
      inv = one / jnp.maximum(zcnt[off + r, :], one)
      for j in range(DH // 16):
        sl = pl.ds(j * 16, 16)
        buf0[r, sl] = buf0[r, sl] * inv
      return c
    lax.fori_loop(0, sz, fin_body, 0)
    pltpu.sync_copy(buf0.at[pl.ds(0, sz)],
                    out_hbm.at[cid, pl.ds(sid * SLICE + off, sz)])


def kernel(data, segment_ids):
  n = data.shape[0]
  ids3d = segment_ids.astype(jnp.int32).reshape(n // 128, 1, 128)

  mesh = plsc.VectorSubcoreMesh(core_axis_name="c", subcore_axis_name="s",
                                num_cores=NC, num_subcores=NS)
  params = pltpu.CompilerParams(use_tc_tiling_on_sc=False)

  sc = pl.kernel(
      _sc_body,
      compiler_params=params,
      out_type=jax.ShapeDtypeStruct((NC, NSEG_PAD, DH), jnp.float32),
      mesh=mesh,
      scratch_types=[
          pltpu.VMEM((SB_ROWS, DH), jnp.float32),
          pltpu.VMEM((SB_ROWS, DH), jnp.float32),
          pltpu.VMEM((SB_ROWS, DH), jnp.float32),
          pltpu.VMEM((NCH, 1, 128), jnp.int32),
          pltpu.VMEM((NCH, 1, 128), jnp.int32),
          pltpu.VMEM((NCH, 1, 128), jnp.int32),
          pltpu.VMEM((CHUNK, 16), jnp.float32),
          pltpu.VMEM((SLICE, 16), jnp.float32),
          pltpu.VMEM_SHARED((NSEG_PAD, DH), jnp.float32),
          pltpu.VMEM_SHARED((NSEG_PAD, 16), jnp.float32),
          pltpu.SemaphoreType.DMA,
          pltpu.SemaphoreType.DMA,
          pltpu.SemaphoreType.DMA,
          pltpu.SemaphoreType.DMA,
      ],
  )
  halves = sc(data, ids3d)
  return jnp.concatenate([halves[0, :NSEG], halves[1, :NSEG]], axis=1)
